# Initial kernel scaffold; baseline (speedup 1.0000x reference)
import functools

import jax
import jax.numpy as jnp
from jax import lax
from jax.experimental import pallas as pl
from jax.experimental.pallas import tpu as pltpu

_TEMP = 0.08838834764831845
_C = 129
_CP = 136
_NT = 1024


def _centers_body(clusters_ref, keys_ref, values_ref, ksum_ref, vsum_ref, cnt_ref):
    g = clusters_ref[0]
    n = g.shape[-1]
    ci = lax.broadcasted_iota(jnp.int32, (_CP, n), 0)
    oh = (ci == g).astype(jnp.float32)
    ksum_ref[0] = lax.dot_general(oh, keys_ref[0], (((1,), (0,)), ((), ())),
                                  preferred_element_type=jnp.float32)
    vsum_ref[0] = lax.dot_general(oh, values_ref[0], (((1,), (0,)), ((), ())),
                                  preferred_element_type=jnp.float32)
    ones = jnp.ones((1, n), jnp.float32)
    cnt_ref[...] = lax.dot_general(ones, oh, (((1,), (1,)), ((), ())),
                                   preferred_element_type=jnp.float32)


def _attn_body(q_ref, ks_ref, vs_ref, cnt_ref, o_ref, a0_ref):
    q = q_ref[0]
    ks = ks_ref[0]
    vs = vs_ref[0]
    cnt = cnt_ref[...]
    lane = lax.broadcasted_iota(jnp.int32, (1, _CP), 1)
    w = jnp.where(lane < _C, _TEMP / cnt, 0.0)
    qk = lax.dot_general(q, ks, (((1,), (1,)), ((), ())),
                         preferred_element_type=jnp.float32)
    qk = qk * w
    m = jnp.max(qk, axis=1, keepdims=True)
    p = jnp.exp(qk - m)
    z = jnp.sum(p * cnt, axis=1, keepdims=True)
    o_ref[0] = lax.dot_general(p, vs, (((1,), (0,)), ((), ())),
                               preferred_element_type=jnp.float32) / z
    a0_ref[0] = p[:, 0:1] * cnt[0, 0] / z


def _centers(keys, values, clusters3):
    b, n, d = keys.shape
    return pl.pallas_call(
        _centers_body,
        grid=(b,),
        in_specs=[
            pl.BlockSpec((1, 1, n), lambda i: (i % 2, 0, 0)),
            pl.BlockSpec((1, n, d), lambda i: (i, 0, 0)),
            pl.BlockSpec((1, n, d), lambda i: (i, 0, 0)),
        ],
        out_specs=[
            pl.BlockSpec((1, _CP, d), lambda i: (i, 0, 0)),
            pl.BlockSpec((1, _CP, d), lambda i: (i, 0, 0)),
            pl.BlockSpec((1, _CP), lambda i: (i, 0)),
        ],
        out_shape=[
            jax.ShapeDtypeStruct((b, _CP, d), jnp.float32),
            jax.ShapeDtypeStruct((b, _CP, d), jnp.float32),
            jax.ShapeDtypeStruct((b, _CP), jnp.float32),
        ],
    )(clusters3, keys, values)


def _attention(queries, ksums, vsums, counts):
    b, n, d = queries.shape
    return pl.pallas_call(
        _attn_body,
        grid=(b, n // _NT),
        in_specs=[
            pl.BlockSpec((1, _NT, d), lambda i, j: (i, j, 0)),
            pl.BlockSpec((1, _CP, d), lambda i, j: (i, 0, 0)),
            pl.BlockSpec((1, _CP, d), lambda i, j: (i, 0, 0)),
            pl.BlockSpec((1, _CP), lambda i, j: (i, 0)),
        ],
        out_specs=[
            pl.BlockSpec((1, _NT, d), lambda i, j: (i, j, 0)),
            pl.BlockSpec((1, _NT, 1), lambda i, j: (i, j, 0)),
        ],
        out_shape=[
            jax.ShapeDtypeStruct((b, n, d), jnp.float32),
            jax.ShapeDtypeStruct((b, n, 1), jnp.float32),
        ],
    )(queries, ksums, vsums, counts)


def kernel(queries, keys, values, clusters):
    b, n, d = queries.shape
    cb = clusters.shape[0]
    clusters3 = clusters.reshape(cb, 1, n)
    ksums, vsums, counts = _centers(keys, values, clusters3)
    v, a0 = _attention(queries, ksums, vsums, counts)
    return v, a0.reshape(b, n)


# speedup vs baseline: 4.5756x; 4.5756x over previous
import functools

import jax
import jax.numpy as jnp
from jax import lax
from jax.experimental import pallas as pl
from jax.experimental.pallas import tpu as pltpu

_TEMP = 0.08838834764831845
_C = 129
_CP = 136
_NT = 1024


def _centers_body(clusters_ref, keys_ref, values_ref, ksum_ref, vsum_ref, cnt_ref):
    g = clusters_ref[0]
    n = g.shape[-1]
    ci = lax.broadcasted_iota(jnp.int32, (_CP, n), 0)
    oh = (ci == g).astype(jnp.float32)
    ksum_ref[0] = lax.dot_general(oh, keys_ref[0], (((1,), (0,)), ((), ())),
                                  preferred_element_type=jnp.float32)
    vsum_ref[0] = lax.dot_general(oh, values_ref[0], (((1,), (0,)), ((), ())),
                                  preferred_element_type=jnp.float32)
    ones = jnp.ones((1, n), jnp.float32)
    cnt_ref[0] = lax.dot_general(ones, oh, (((1,), (1,)), ((), ())),
                                 preferred_element_type=jnp.float32)


def _attn_body(q_ref, ks_ref, vs_ref, cnt_ref, o_ref, a0_ref):
    q = q_ref[0]
    ks = ks_ref[0]
    vs = vs_ref[0]
    cnt = cnt_ref[0]
    lane = lax.broadcasted_iota(jnp.int32, (1, _CP), 1)
    w = jnp.where(lane < _C, _TEMP / cnt, 0.0)
    qk = lax.dot_general(q, ks, (((1,), (1,)), ((), ())),
                         preferred_element_type=jnp.float32)
    qk = qk * w
    m = jnp.max(qk, axis=1, keepdims=True)
    p = jnp.exp(qk - m)
    z = jnp.sum(p * cnt, axis=1, keepdims=True)
    o_ref[0] = lax.dot_general(p, vs, (((1,), (0,)), ((), ())),
                               preferred_element_type=jnp.float32) / z
    a0_ref[0] = p[:, 0:1] * cnt[0, 0] / z


def _centers(keys, values, clusters3):
    b, n, d = keys.shape
    return pl.pallas_call(
        _centers_body,
        grid=(b,),
        in_specs=[
            pl.BlockSpec((1, 1, n), lambda i: (i % 2, 0, 0)),
            pl.BlockSpec((1, n, d), lambda i: (i, 0, 0)),
            pl.BlockSpec((1, n, d), lambda i: (i, 0, 0)),
        ],
        out_specs=[
            pl.BlockSpec((1, _CP, d), lambda i: (i, 0, 0)),
            pl.BlockSpec((1, _CP, d), lambda i: (i, 0, 0)),
            pl.BlockSpec((1, 1, _CP), lambda i: (i, 0, 0)),
        ],
        out_shape=[
            jax.ShapeDtypeStruct((b, _CP, d), jnp.float32),
            jax.ShapeDtypeStruct((b, _CP, d), jnp.float32),
            jax.ShapeDtypeStruct((b, 1, _CP), jnp.float32),
        ],
    )(clusters3, keys, values)


def _attention(queries, ksums, vsums, counts):
    b, n, d = queries.shape
    return pl.pallas_call(
        _attn_body,
        grid=(b, n // _NT),
        in_specs=[
            pl.BlockSpec((1, _NT, d), lambda i, j: (i, j, 0)),
            pl.BlockSpec((1, _CP, d), lambda i, j: (i, 0, 0)),
            pl.BlockSpec((1, _CP, d), lambda i, j: (i, 0, 0)),
            pl.BlockSpec((1, 1, _CP), lambda i, j: (i, 0, 0)),
        ],
        out_specs=[
            pl.BlockSpec((1, _NT, d), lambda i, j: (i, j, 0)),
            pl.BlockSpec((1, _NT, 1), lambda i, j: (i, j, 0)),
        ],
        out_shape=[
            jax.ShapeDtypeStruct((b, n, d), jnp.float32),
            jax.ShapeDtypeStruct((b, n, 1), jnp.float32),
        ],
    )(queries, ksums, vsums, counts)


def kernel(queries, keys, values, clusters):
    b, n, d = queries.shape
    cb = clusters.shape[0]
    clusters3 = clusters.reshape(cb, 1, n)
    ksums, vsums, counts = _centers(keys, values, clusters3)
    v, a0 = _attention(queries, ksums, vsums, counts)
    return v, a0.reshape(b, n)
